# Initial kernel scaffold; baseline (speedup 1.0000x reference)
import jax
import jax.numpy as jnp
from jax.experimental import pallas as pl


def kernel(x1, x2, bn1, bn2):
    raise NotImplementedError("write your pallas kernel here")


# speedup vs baseline: 6.7920x; 6.7920x over previous
import functools

import jax
import jax.numpy as jnp
from jax import lax
from jax.experimental import pallas as pl
from jax.experimental.pallas import tpu as pltpu
from jax.experimental.pallas import tpu_sc as plsc

C = 2048
K = 1024
NROW = 2 * C
S = 4096
NW = 32
CHUNK = 8
NCHUNK = 8
RCH = 256


def _rank_lt_k(b_row, b_col):
    cols = []
    for ci in range(C // RCH):
        bi = lax.slice(b_col, (ci * RCH, 0), ((ci + 1) * RCH, 1))
        fi = lax.broadcasted_iota(jnp.int32, (RCH, C), 0) + ci * RCH
        fj = lax.broadcasted_iota(jnp.int32, (RCH, C), 1)
        beat = (b_row > bi) | ((b_row == bi) & (fj < fi))
        rank = jnp.sum(beat.astype(jnp.int32), axis=1, keepdims=True)
        cols.append((rank < K).astype(jnp.int32))
    return jnp.concatenate(cols, axis=0)


def _inclusive_cumsum(m_col):
    m_row = jnp.transpose(m_col)
    outs = []
    for ci in range(C // RCH):
        fi = lax.broadcasted_iota(jnp.int32, (RCH, C), 0) + ci * RCH
        fj = lax.broadcasted_iota(jnp.int32, (RCH, C), 1)
        tri = (fj <= fi).astype(jnp.int32)
        outs.append(jnp.sum(tri * m_row, axis=1, keepdims=True))
    return jnp.concatenate(outs, axis=0)


def _compact(m_col):
    pos = _inclusive_cumsum(m_col) - 1
    ks = lax.broadcasted_iota(jnp.int32, (C, K), 1)
    fi = lax.broadcasted_iota(jnp.int32, (C, K), 0)
    onehot = (pos == ks) & (m_col == 1)
    return jnp.sum(jnp.where(onehot, fi, 0), axis=0, keepdims=True)


def _index_kernel(b1r_ref, b1c_ref, b2r_ref, b2c_ref, src_ref, dst_ref):
    b1r = jnp.abs(b1r_ref[...])
    b1c = jnp.abs(b1c_ref[...])
    b2r = jnp.abs(b2r_ref[...])
    b2c = jnp.abs(b2c_ref[...])

    id1 = _rank_lt_k(b1r, b1c)
    id2 = _rank_lt_k(b2r, b2c)

    i1t = _compact(id1)
    i1f = _compact(1 - id1)
    i2t = _compact(id2)
    i2f = _compact(1 - id2)

    src_ref[...] = jnp.concatenate(
        [i1t, i1t + C, i2f, i2f + C, i2t, i2t + C, i1f, i1f + C], axis=0)
    dst_ref[...] = jnp.concatenate(
        [i1t, i1t + C, i1f, i1f + C, i2t, i2t + C, i2f, i2f + C], axis=0)


def _build_rows(bn1, bn2):
    return pl.pallas_call(
        _index_kernel,
        out_shape=(jax.ShapeDtypeStruct((8, K), jnp.int32),
                   jax.ShapeDtypeStruct((8, K), jnp.int32)),
    )(bn1.reshape(1, C), bn1.reshape(C, 1),
      bn2.reshape(1, C), bn2.reshape(C, 1))


@functools.cache
def _make_exchange_sc():
    mesh = plsc.VectorSubcoreMesh(core_axis_name="c", subcore_axis_name="s")
    return pl.kernel(
        _exchange_sc_body,
        out_type=(jax.ShapeDtypeStruct((NROW, S), jnp.float32),
                  jax.ShapeDtypeStruct((NROW, S), jnp.float32)),
        mesh=mesh,
        scratch_types=[
            pltpu.VMEM((4, NCHUNK, CHUNK), jnp.int32),
            pltpu.VMEM((4, NCHUNK, CHUNK), jnp.int32),
            pltpu.VMEM((CHUNK, S), jnp.float32),
            pltpu.SemaphoreType.DMA,
            pltpu.SemaphoreType.DMA,
        ],
    )


def _exchange_sc_body(x1h, x2h, srch, dsth, y1h, y2h,
                      sidx, didx, buf0, gs0, ss0):
    wid = lax.axis_index("s") * 2 + lax.axis_index("c")
    pltpu.sync_copy(srch.at[wid], sidx)
    pltpu.sync_copy(dsth.at[wid], didx)

    srcs = (x1h, x2h, x2h, x1h)
    dsts = (y1h, y1h, y2h, y2h)
    for g in range(4):
        def body(j, carry, g=g):
            pltpu.async_copy(srcs[g].at[sidx.at[g, j]], buf0, gs0).wait()
            pltpu.async_copy(buf0, dsts[g].at[didx.at[g, j]], ss0).wait()
            return carry
        lax.fori_loop(0, NCHUNK, body, 0)


def kernel(x1, x2, bn1, bn2):
    B = x1.shape[0]
    src_rows, dst_rows = _build_rows(bn1, bn2)
    src_w = jnp.transpose(
        src_rows.reshape(4, NW, NCHUNK, CHUNK), (1, 0, 2, 3))
    dst_w = jnp.transpose(
        dst_rows.reshape(4, NW, NCHUNK, CHUNK), (1, 0, 2, 3))
    y1, y2 = _make_exchange_sc()(x1.reshape(NROW, S), x2.reshape(NROW, S),
                                 src_w, dst_w)
    return y1.reshape(B, C, S), y2.reshape(B, C, S)


# speedup vs baseline: 7.8054x; 1.1492x over previous
import functools

import jax
import jax.numpy as jnp
from jax import lax
from jax.experimental import pallas as pl
from jax.experimental.pallas import tpu as pltpu
from jax.experimental.pallas import tpu_sc as plsc

C = 2048
K = 1024
NROW = 2 * C
S = 4096
NW = 32
CHUNK = 8
NCHUNK = 8
RCH = 256


def _rank_lt_k(b_row, b_col):
    cols = []
    for ci in range(C // RCH):
        bi = lax.slice(b_col, (ci * RCH, 0), ((ci + 1) * RCH, 1))
        fi = lax.broadcasted_iota(jnp.int32, (RCH, C), 0) + ci * RCH
        fj = lax.broadcasted_iota(jnp.int32, (RCH, C), 1)
        beat = (b_row > bi) | ((b_row == bi) & (fj < fi))
        rank = jnp.sum(beat.astype(jnp.int32), axis=1, keepdims=True)
        cols.append((rank < K).astype(jnp.int32))
    return jnp.concatenate(cols, axis=0)


def _inclusive_cumsum(m_col):
    m_row = jnp.transpose(m_col)
    outs = []
    for ci in range(C // RCH):
        fi = lax.broadcasted_iota(jnp.int32, (RCH, C), 0) + ci * RCH
        fj = lax.broadcasted_iota(jnp.int32, (RCH, C), 1)
        tri = (fj <= fi).astype(jnp.int32)
        outs.append(jnp.sum(tri * m_row, axis=1, keepdims=True))
    return jnp.concatenate(outs, axis=0)


def _compact(m_col):
    pos = _inclusive_cumsum(m_col) - 1
    ks = lax.broadcasted_iota(jnp.int32, (C, K), 1)
    fi = lax.broadcasted_iota(jnp.int32, (C, K), 0)
    onehot = (pos == ks) & (m_col == 1)
    return jnp.sum(jnp.where(onehot, fi, 0), axis=0, keepdims=True)


def _index_kernel(b1r_ref, b1c_ref, b2r_ref, b2c_ref, src_ref, dst_ref):
    b1r = jnp.abs(b1r_ref[...])
    b1c = jnp.abs(b1c_ref[...])
    b2r = jnp.abs(b2r_ref[...])
    b2c = jnp.abs(b2c_ref[...])

    id1 = _rank_lt_k(b1r, b1c)
    id2 = _rank_lt_k(b2r, b2c)

    i1t = _compact(id1)
    i1f = _compact(1 - id1)
    i2t = _compact(id2)
    i2f = _compact(1 - id2)

    src_ref[...] = jnp.concatenate(
        [i1t, i1t + C, i2f, i2f + C, i2t, i2t + C, i1f, i1f + C], axis=0)
    dst_ref[...] = jnp.concatenate(
        [i1t, i1t + C, i1f, i1f + C, i2t, i2t + C, i2f, i2f + C], axis=0)


def _build_rows(bn1, bn2):
    return pl.pallas_call(
        _index_kernel,
        out_shape=(jax.ShapeDtypeStruct((8, K), jnp.int32),
                   jax.ShapeDtypeStruct((8, K), jnp.int32)),
    )(bn1.reshape(1, C), bn1.reshape(C, 1),
      bn2.reshape(1, C), bn2.reshape(C, 1))


@functools.cache
def _make_exchange_sc():
    mesh = plsc.VectorSubcoreMesh(core_axis_name="c", subcore_axis_name="s")
    return pl.kernel(
        _exchange_sc_body,
        out_type=(jax.ShapeDtypeStruct((NROW, S), jnp.float32),
                  jax.ShapeDtypeStruct((NROW, S), jnp.float32)),
        mesh=mesh,
        scratch_types=[
            pltpu.VMEM((4, NCHUNK, CHUNK), jnp.int32),
            pltpu.VMEM((4, NCHUNK, CHUNK), jnp.int32),
            pltpu.VMEM((CHUNK, S), jnp.float32),
            pltpu.VMEM((CHUNK, S), jnp.float32),
            pltpu.SemaphoreType.DMA,
            pltpu.SemaphoreType.DMA,
            pltpu.SemaphoreType.DMA,
            pltpu.SemaphoreType.DMA,
        ],
    )


def _exchange_sc_body(x1h, x2h, srch, dsth, y1h, y2h,
                      sidx, didx, buf0, buf1, gs0, gs1, ss0, ss1):
    wid = lax.axis_index("s") * 2 + lax.axis_index("c")
    pltpu.sync_copy(srch.at[wid], sidx)
    pltpu.sync_copy(dsth.at[wid], didx)

    srcs = (x1h, x2h, x2h, x1h)
    dsts = (y1h, y1h, y2h, y2h)
    bufs = (buf0, buf1)
    gsems = (gs0, gs1)
    ssems = (ss0, ss1)

    def gather(g, j, b):
        pltpu.async_copy(srcs[g].at[sidx.at[g, j]], bufs[b], gsems[b])

    def gather_wait(g, j, b):
        pltpu.make_async_copy(srcs[g].at[sidx.at[g, j]], bufs[b],
                              gsems[b]).wait()

    def scatter(g, j, b):
        pltpu.async_copy(bufs[b], dsts[g].at[didx.at[g, j]], ssems[b])

    def scatter_wait(g, j, b):
        pltpu.make_async_copy(bufs[b], dsts[g].at[didx.at[g, j]],
                              ssems[b]).wait()

    for g in range(4):
        gather(g, 0, 0)
        gather(g, 1, 1)

        def body(p, carry, g=g):
            j = 2 * p
            gather_wait(g, j, 0)
            scatter(g, j, 0)
            scatter_wait(g, j, 0)
            gather(g, j + 2, 0)
            gather_wait(g, j + 1, 1)
            scatter(g, j + 1, 1)
            scatter_wait(g, j + 1, 1)
            gather(g, j + 3, 1)
            return carry

        lax.fori_loop(0, (NCHUNK - 2) // 2, body, 0)
        gather_wait(g, NCHUNK - 2, 0)
        scatter(g, NCHUNK - 2, 0)
        gather_wait(g, NCHUNK - 1, 1)
        scatter(g, NCHUNK - 1, 1)
        scatter_wait(g, NCHUNK - 2, 0)
        scatter_wait(g, NCHUNK - 1, 1)


def kernel(x1, x2, bn1, bn2):
    B = x1.shape[0]
    src_rows, dst_rows = _build_rows(bn1, bn2)
    src_w = jnp.transpose(
        src_rows.reshape(4, NW, NCHUNK, CHUNK), (1, 0, 2, 3))
    dst_w = jnp.transpose(
        dst_rows.reshape(4, NW, NCHUNK, CHUNK), (1, 0, 2, 3))
    y1, y2 = _make_exchange_sc()(x1.reshape(NROW, S), x2.reshape(NROW, S),
                                 src_w, dst_w)
    return y1.reshape(B, C, S), y2.reshape(B, C, S)
